# baseline (speedup 1.0000x reference)
import jax
import jax.numpy as jnp
from jax.experimental import pallas as pl
from jax.experimental.pallas import tpu as pltpu

N = 10000
F_IN = 128
F_OUT = 128
BM = 400


def _body(adj_ref, feat_ref, w_ref, out_ref, fe_ref):
    @pl.when(pl.program_id(0) == 0)
    def _():
        fe_ref[...] = jnp.dot(feat_ref[...], w_ref[...],
                              preferred_element_type=jnp.float32)

    out_ref[...] = jnp.dot(adj_ref[...], fe_ref[...],
                           preferred_element_type=jnp.float32)


def kernel(feat, adj, weight):
    grid = (N // BM,)
    return pl.pallas_call(
        _body,
        grid=grid,
        in_specs=[
            pl.BlockSpec((BM, N), lambda i: (i, 0)),
            pl.BlockSpec((N, F_IN), lambda i: (0, 0)),
            pl.BlockSpec((F_IN, F_OUT), lambda i: (0, 0)),
        ],
        out_specs=pl.BlockSpec((BM, F_OUT), lambda i: (i, 0)),
        out_shape=jax.ShapeDtypeStruct((N, F_OUT), jnp.float32),
        scratch_shapes=[pltpu.VMEM((N, F_OUT), jnp.float32)],
    )(adj, feat, weight)


# speedup vs baseline: 1.0105x; 1.0105x over previous
import jax
import jax.numpy as jnp
from jax.experimental import pallas as pl
from jax.experimental.pallas import tpu as pltpu

N = 10000
F_IN = 128
F_OUT = 128
BM = 200


def _body(adj_ref, feat_ref, w_ref, out_ref, fe_ref):
    @pl.when(pl.program_id(0) == 0)
    def _():
        fe_ref[...] = jnp.dot(feat_ref[...], w_ref[...],
                              preferred_element_type=jnp.float32)

    out_ref[...] = jnp.dot(adj_ref[...], fe_ref[...],
                           preferred_element_type=jnp.float32)


def kernel(feat, adj, weight):
    grid = (N // BM,)
    return pl.pallas_call(
        _body,
        grid=grid,
        in_specs=[
            pl.BlockSpec((BM, N), lambda i: (i, 0)),
            pl.BlockSpec((N, F_IN), lambda i: (0, 0)),
            pl.BlockSpec((F_IN, F_OUT), lambda i: (0, 0)),
        ],
        out_specs=pl.BlockSpec((BM, F_OUT), lambda i: (i, 0)),
        out_shape=jax.ShapeDtypeStruct((N, F_OUT), jnp.float32),
        scratch_shapes=[pltpu.VMEM((N, F_OUT), jnp.float32)],
    )(adj, feat, weight)


# speedup vs baseline: 1.0122x; 1.0017x over previous
import jax
import jax.numpy as jnp
from jax.experimental import pallas as pl
from jax.experimental.pallas import tpu as pltpu

N = 10000
F_IN = 128
F_OUT = 128
BM = 400


def _body(adj_ref, feat_ref, w_ref, out_ref, fe_ref):
    @pl.when(pl.program_id(0) == 0)
    def _():
        fe_ref[...] = jnp.dot(feat_ref[...], w_ref[...],
                              preferred_element_type=jnp.float32)

    out_ref[...] = jnp.dot(adj_ref[...], fe_ref[...],
                           preferred_element_type=jnp.float32)


def kernel(feat, adj, weight):
    grid = (N // BM,)
    return pl.pallas_call(
        _body,
        grid=grid,
        in_specs=[
            pl.BlockSpec((BM, N), lambda i: (i, 0)),
            pl.BlockSpec((N, F_IN), lambda i: (0, 0)),
            pl.BlockSpec((F_IN, F_OUT), lambda i: (0, 0)),
        ],
        out_specs=pl.BlockSpec((BM, F_OUT), lambda i: (i, 0)),
        out_shape=jax.ShapeDtypeStruct((N, F_OUT), jnp.float32),
        scratch_shapes=[pltpu.VMEM((N, F_OUT), jnp.float32)],
        compiler_params=pltpu.CompilerParams(
            vmem_limit_bytes=120 * 1024 * 1024),
    )(adj, feat, weight)
